# Initial kernel scaffold; baseline (speedup 1.0000x reference)
import jax
import jax.numpy as jnp
from jax import lax
from jax.experimental import pallas as pl
from jax.experimental.pallas import tpu as pltpu
from jax.experimental.pallas import tpu_sc as plsc

B = 16384
MOVIE_DIM = 64
CAT_DIM = 32
OUT_DIM = MOVIE_DIM + CAT_DIM
L = 10
NC = 2
NS = 16
NW = NC * NS
BPW = B // NW
CH = 128
NCH = BPW // CH
LANES = 16


def _body(mid_hbm, mct_hbm, movies_hbm, cats_hbm, bias_hbm,
          out_vec_hbm, out_bias_hbm,
          mid_v, cats_v, mrows_v, bias_v, g_v, recip_v, outc_v,
          sem_m, sem_b, sem_c):
    cid = lax.axis_index("c")
    sid = lax.axis_index("s")
    wid = sid * NC + cid
    base = wid * BPW

    pltpu.sync_copy(mid_hbm.at[wid], mid_v)
    pltpu.sync_copy(mct_hbm.at[wid], cats_v)

    movie_dmas = [
        pltpu.async_copy(movies_hbm.at[mid_v.at[c]], mrows_v.at[c], sem_m)
        for c in range(NCH)
    ]
    bias_dmas = [
        pltpu.async_copy(bias_hbm.at[mid_v.at[c]], bias_v.at[c], sem_b)
        for c in range(NCH)
    ]

    lanes = lax.iota(jnp.int32, LANES)

    for c in range(NCH):
        cat_dmas = [
            pltpu.async_copy(cats_hbm.at[cats_v.at[l, c]], g_v.at[l], sem_c)
            for l in range(L)
        ]

        def cnt_body(g, carry):
            cnt = jnp.zeros((LANES,), jnp.float32)
            for l in range(L):
                cv = cats_v[l, c, pl.ds(g * LANES, LANES)]
                cnt = cnt + jnp.where(cv != 0, 1.0, 0.0)
            recip = 1.0 / jnp.maximum(cnt, 1.0)
            idx0 = lanes * CAT_DIM + g * (LANES * CAT_DIM)
            for dcol in range(CAT_DIM):
                plsc.store_scatter(recip_v, [idx0 + dcol], recip)
            return carry

        lax.fori_loop(0, CH // LANES, cnt_body, 0)

        for dma in cat_dmas:
            dma.wait()

        def row_body(r, carry):
            for h in range(CAT_DIM // LANES):
                s = g_v[0, r, pl.ds(h * LANES, LANES)]
                for l in range(1, L):
                    s = s + g_v[l, r, pl.ds(h * LANES, LANES)]
                rp = recip_v[pl.ds(r * CAT_DIM + h * LANES, LANES)]
                outc_v[r, pl.ds(h * LANES, LANES)] = s * rp
            return carry

        lax.fori_loop(0, CH, row_body, 0)

        pltpu.sync_copy(
            outc_v,
            out_vec_hbm.at[pl.ds(base + c * CH, CH), pl.ds(MOVIE_DIM, CAT_DIM)])

    for c in range(NCH):
        movie_dmas[c].wait()
        pltpu.sync_copy(
            mrows_v.at[c],
            out_vec_hbm.at[pl.ds(base + c * CH, CH), pl.ds(0, MOVIE_DIM)])
    for c in range(NCH):
        bias_dmas[c].wait()
        pltpu.sync_copy(bias_v.at[c], out_bias_hbm.at[pl.ds(base + c * CH, CH)])


_sc_call = pl.kernel(
    _body,
    out_type=[
        jax.ShapeDtypeStruct((B, OUT_DIM), jnp.float32),
        jax.ShapeDtypeStruct((B, 1), jnp.float32),
    ],
    mesh=plsc.VectorSubcoreMesh(core_axis_name="c", subcore_axis_name="s"),
    scratch_types=[
        pltpu.VMEM((NCH, CH), jnp.int32),
        pltpu.VMEM((L, NCH, CH), jnp.int32),
        pltpu.VMEM((NCH, CH, MOVIE_DIM), jnp.float32),
        pltpu.VMEM((NCH, CH, 1), jnp.float32),
        pltpu.VMEM((L, CH, CAT_DIM), jnp.float32),
        pltpu.VMEM((CH * CAT_DIM,), jnp.float32),
        pltpu.VMEM((CH, CAT_DIM), jnp.float32),
        pltpu.SemaphoreType.DMA,
        pltpu.SemaphoreType.DMA,
        pltpu.SemaphoreType.DMA,
    ],
)


def kernel(movie_id, movie_categories, emb_movies_W, emb_cats_W, bias_W):
    mid = movie_id.astype(jnp.int32).reshape(NW, NCH, CH)
    mct = (movie_categories.astype(jnp.int32).T
           .reshape(L, NW, NCH, CH).transpose(1, 0, 2, 3))
    vec, bias = _sc_call(mid, mct, emb_movies_W, emb_cats_W, bias_W)
    return vec, bias.reshape(-1)


# speedup vs baseline: 1.2812x; 1.2812x over previous
import jax
import jax.numpy as jnp
from jax import lax
from jax.experimental import pallas as pl
from jax.experimental.pallas import tpu as pltpu
from jax.experimental.pallas import tpu_sc as plsc

B = 16384
MOVIE_DIM = 64
CAT_DIM = 32
OUT_DIM = MOVIE_DIM + CAT_DIM
L = 10
NC = 2
NS = 16
NW = NC * NS
BPW = B // NW
CH = 128
NCH = BPW // CH
CHC = 16
NQ = CH // CHC
LANES = 16


def _body(mid_hbm, mct_hbm, movies_hbm, cats_hbm, bias_hbm,
          out_vec_hbm, out_bias_hbm,
          mid_v, cats_v, mrows_v, g_v, comb_v, bhi_v, bias_g, bias_c,
          sem_m0, sem_m1, sem_b, sem_c):
    cid = lax.axis_index("c")
    sid = lax.axis_index("s")
    wid = sid * NC + cid
    base = wid * BPW

    pltpu.sync_copy(mid_hbm.at[wid], mid_v)
    pltpu.sync_copy(mct_hbm.at[wid], cats_v)

    def issue_movie(c):
        return pltpu.async_copy(
            movies_hbm.at[mid_v.at[pl.ds(c * CH, CH)]],
            mrows_v.at[c % 2], sem_m0 if c % 2 == 0 else sem_m1)

    movie_dmas = [issue_movie(0), issue_movie(1)]

    def hi_body(i, carry):
        bhi_v[pl.ds(i * LANES, LANES)] = (
            lax.shift_right_logical(mid_v[pl.ds(i * LANES, LANES)], 4))
        return carry

    lax.fori_loop(0, BPW // LANES, hi_body, 0)

    lane_iota = lax.iota(jnp.int32, LANES)

    for c in range(NCH):
        bias_dma = pltpu.async_copy(
            bias_hbm.at[bhi_v.at[pl.ds(c * CH, CH)]], bias_g, sem_b)

        def q_body(q, carry):
            cat_dmas = [
                pltpu.async_copy(
                    cats_hbm.at[cats_v.at[pl.ds((c * L + l) * CH + q * CHC,
                                                CHC)]],
                    g_v.at[l], sem_c)
                for l in range(L)
            ]
            for dma in cat_dmas:
                dma.wait()

            cnt = jnp.zeros((LANES,), jnp.float32)
            for l in range(L):
                cv = cats_v[pl.ds((c * L + l) * CH + q * CHC, LANES)]
                cnt = cnt + jnp.where(cv != 0, 1.0, 0.0)
            recip = 1.0 / jnp.maximum(cnt, 1.0)
            for j in range(LANES):
                rp = recip[j]
                for h in range(CAT_DIM // LANES):
                    s = g_v[0, j, pl.ds(h * LANES, LANES)]
                    for l in range(1, L):
                        s = s + g_v[l, j, pl.ds(h * LANES, LANES)]
                    comb_v[q * CHC + j,
                           pl.ds(MOVIE_DIM + h * LANES, LANES)] = s * rp
            return carry

        lax.fori_loop(0, NQ, q_body, 0)

        movie_dmas[c].wait()

        def cp_body(r, carry):
            for h in range(MOVIE_DIM // LANES):
                comb_v[r, pl.ds(h * LANES, LANES)] = (
                    mrows_v[c % 2, r, pl.ds(h * LANES, LANES)])
            return carry

        lax.fori_loop(0, CH, cp_body, 0)

        if c + 2 < NCH:
            movie_dmas.append(issue_movie(c + 2))

        pltpu.sync_copy(comb_v, out_vec_hbm.at[pl.ds(base + c * CH, CH)])

        bias_dma.wait()

        def bias_body(g, carry):
            lanes16 = mid_v[pl.ds(c * CH + g * LANES, LANES)] & 15
            vals = jnp.zeros((LANES,), jnp.float32)
            for j in range(LANES):
                row = bias_g[g * LANES + j, pl.ds(0, LANES)]
                picked = jnp.take_along_axis(row, lanes16, axis=0)
                vals = jnp.where(lane_iota == j, picked, vals)
            bias_c[pl.ds(g * LANES, LANES)] = vals
            return carry

        lax.fori_loop(0, CH // LANES, bias_body, 0)
        pltpu.sync_copy(bias_c, out_bias_hbm.at[pl.ds(base + c * CH, CH)])


_sc_call = pl.kernel(
    _body,
    out_type=[
        jax.ShapeDtypeStruct((B, OUT_DIM), jnp.float32),
        jax.ShapeDtypeStruct((B,), jnp.float32),
    ],
    mesh=plsc.VectorSubcoreMesh(core_axis_name="c", subcore_axis_name="s"),
    compiler_params=pltpu.CompilerParams(use_tc_tiling_on_sc=False),
    scratch_types=[
        pltpu.VMEM((BPW,), jnp.int32),
        pltpu.VMEM((NCH * L * CH,), jnp.int32),
        pltpu.VMEM((2, CH, MOVIE_DIM), jnp.float32),
        pltpu.VMEM((L, CHC, CAT_DIM), jnp.float32),
        pltpu.VMEM((CH, OUT_DIM), jnp.float32),
        pltpu.VMEM((BPW,), jnp.int32),
        pltpu.VMEM((CH, 16), jnp.float32),
        pltpu.VMEM((CH,), jnp.float32),
        pltpu.SemaphoreType.DMA,
        pltpu.SemaphoreType.DMA,
        pltpu.SemaphoreType.DMA,
        pltpu.SemaphoreType.DMA,
    ],
)


def kernel(movie_id, movie_categories, emb_movies_W, emb_cats_W, bias_W):
    mid = movie_id.astype(jnp.int32).reshape(NW, BPW)
    mct = (movie_categories.astype(jnp.int32).T
           .reshape(L, NW, NCH, CH).transpose(1, 2, 0, 3)
           .reshape(NW, NCH * L * CH))
    bias16 = bias_W.reshape(-1, 16)
    vec, bias = _sc_call(mid, mct, emb_movies_W, emb_cats_W, bias16)
    return vec, bias


# speedup vs baseline: 1.2821x; 1.0007x over previous
import jax
import jax.numpy as jnp
from jax import lax
from jax.experimental import pallas as pl
from jax.experimental.pallas import tpu as pltpu
from jax.experimental.pallas import tpu_sc as plsc

B = 16384
MOVIE_DIM = 64
CAT_DIM = 32
OUT_DIM = MOVIE_DIM + CAT_DIM
L = 10
NC = 2
NS = 16
NW = NC * NS
BPW = B // NW
CH = 128
NCH = BPW // CH
CHC = 16
NQ = CH // CHC
LANES = 16


def _body(mid_hbm, mct_hbm, movies_hbm, cats_hbm, bias_hbm,
          out_vec_hbm, out_bias_hbm,
          mid_v, cats_v, mrows_v, g_v, comb_v, bhi_v, bias_g, bias_c,
          sem_m0, sem_m1, sem_b, sem_c):
    cid = lax.axis_index("c")
    sid = lax.axis_index("s")
    wid = sid * NC + cid
    base = wid * BPW

    pltpu.sync_copy(mid_hbm.at[wid], mid_v)
    pltpu.sync_copy(mct_hbm.at[wid], cats_v)

    def issue_movie(c):
        return pltpu.async_copy(
            movies_hbm.at[mid_v.at[pl.ds(c * CH, CH)]],
            mrows_v.at[c % 2], sem_m0 if c % 2 == 0 else sem_m1)

    movie_dmas = [issue_movie(0), issue_movie(1)]

    def hi_body(i, carry):
        bhi_v[pl.ds(i * LANES, LANES)] = (
            lax.shift_right_logical(mid_v[pl.ds(i * LANES, LANES)], 4))
        return carry

    lax.fori_loop(0, BPW // LANES, hi_body, 0)

    lane_iota = lax.iota(jnp.int32, LANES)

    for c in range(NCH):
        bias_dma = pltpu.async_copy(
            bias_hbm.at[bhi_v.at[pl.ds(c * CH, CH)]], bias_g, sem_b)

        def q_body(q, carry):
            cat_dmas = [
                pltpu.async_copy(
                    cats_hbm.at[cats_v.at[pl.ds((c * L + l) * CH + q * CHC,
                                                CHC)]],
                    g_v.at[l], sem_c)
                for l in range(L)
            ]
            for dma in cat_dmas:
                dma.wait()

            cnt = jnp.zeros((LANES,), jnp.float32)
            for l in range(L):
                cv = cats_v[pl.ds((c * L + l) * CH + q * CHC, LANES)]
                cnt = cnt + jnp.where(cv != 0, 1.0, 0.0)
            recip = 1.0 / jnp.maximum(cnt, 1.0)
            for j in range(LANES):
                rp = recip[j]
                for h in range(CAT_DIM // LANES):
                    s = g_v[0, j, pl.ds(h * LANES, LANES)]
                    for l in range(1, L):
                        s = s + g_v[l, j, pl.ds(h * LANES, LANES)]
                    comb_v[q * CHC + j,
                           pl.ds(MOVIE_DIM + h * LANES, LANES)] = s * rp
            return carry

        lax.fori_loop(0, NQ, q_body, 0)

        movie_dmas[c].wait()

        def cp_body(r, carry):
            for h in range(MOVIE_DIM // LANES):
                comb_v[r, pl.ds(h * LANES, LANES)] = (
                    mrows_v[c % 2, r, pl.ds(h * LANES, LANES)])
            return carry

        lax.fori_loop(0, CH, cp_body, 0)

        if c + 2 < NCH:
            movie_dmas.append(issue_movie(c + 2))

        pltpu.sync_copy(comb_v, out_vec_hbm.at[pl.ds(base + c * CH, CH)])

        bias_dma.wait()

        def bias_body(g, carry):
            lanes16 = mid_v[pl.ds(c * CH + g * LANES, LANES)] & 15
            vals = jnp.zeros((LANES,), jnp.float32)
            for j in range(LANES):
                row = bias_g[g * LANES + j, pl.ds(0, LANES)]
                picked = jnp.take_along_axis(row, lanes16, axis=0)
                vals = jnp.where(lane_iota == j, picked, vals)
            bias_c[pl.ds(g * LANES, LANES)] = vals
            return carry

        lax.fori_loop(0, CH // LANES, bias_body, 0)
        pltpu.sync_copy(bias_c, out_bias_hbm.at[pl.ds(base + c * CH, CH)])


_sc_call = pl.kernel(
    _body,
    out_type=[
        jax.ShapeDtypeStruct((B, OUT_DIM), jnp.float32),
        jax.ShapeDtypeStruct((B,), jnp.float32),
    ],
    mesh=plsc.VectorSubcoreMesh(core_axis_name="c", subcore_axis_name="s"),
    compiler_params=pltpu.CompilerParams(use_tc_tiling_on_sc=False),
    scratch_types=[
        pltpu.VMEM((BPW,), jnp.int32),
        pltpu.VMEM((NCH * L * CH,), jnp.int32),
        pltpu.VMEM((2, CH, MOVIE_DIM), jnp.float32),
        pltpu.VMEM((L, CHC, CAT_DIM), jnp.float32),
        pltpu.VMEM((CH, OUT_DIM), jnp.float32),
        pltpu.VMEM((BPW,), jnp.int32),
        pltpu.VMEM((CH, 16), jnp.float32),
        pltpu.VMEM((CH,), jnp.float32),
        pltpu.SemaphoreType.DMA,
        pltpu.SemaphoreType.DMA,
        pltpu.SemaphoreType.DMA,
        pltpu.SemaphoreType.DMA,
    ],
)


def kernel(movie_id, movie_categories, emb_movies_W, emb_cats_W, bias_W):
    mid = movie_id.astype(jnp.int32).reshape(NW, BPW)
    mct = (movie_categories.astype(jnp.int32).T
           .reshape(L, NW, NCH, CH).transpose(1, 2, 0, 3)
           .reshape(NW, NCH * L * CH))
    bias16 = bias_W.T.reshape(1000000 // 16, 16)
    vec, bias = _sc_call(mid, mct, emb_movies_W, emb_cats_W, bias16)
    return vec, bias


# speedup vs baseline: 1.9036x; 1.4847x over previous
import jax
import jax.numpy as jnp
from jax import lax
from jax.experimental import pallas as pl
from jax.experimental.pallas import tpu as pltpu
from jax.experimental.pallas import tpu_sc as plsc

B = 16384
NUM_MOVIES = 1000000
MOVIE_DIM = 64
CAT_DIM = 32
L = 10
NC = 2
NS = 16
NW = NC * NS
BPW = B // NW
CH = 128
NCH = BPW // CH
CHC = 16
NQ = CH // CHC
LANES = 16
NG = BPW // LANES


def _movie_body(mid_hbm, movies_hbm, out_hbm, mid_v, grp_v, comb_v,
                sem_g0, sem_g1):
    cid = lax.axis_index("c")
    sid = lax.axis_index("s")
    wid = sid * NC + cid
    base = wid * BPW

    pltpu.sync_copy(mid_hbm.at[wid], mid_v)
    sems = (sem_g0, sem_g1)

    def issue_group(g, p):
        mv = mid_v[g // 8, pl.ds((g % 8) * LANES, LANES)]
        for j in range(LANES):
            idx = mv[j]
            g8 = pl.multiple_of((idx // 8) * 8, 8)
            pltpu.async_copy(movies_hbm.at[pl.ds(g8, 8)],
                             grp_v.at[p, j], sems[p])

    issue_group(0, 0)
    issue_group(1, 1)

    def g2_body(k, carry):
        for p in range(2):
            g = 2 * k + p
            mv = mid_v[g // 8, pl.ds((g % 8) * LANES, LANES)]
            for j in range(LANES):
                pltpu.make_async_copy(movies_hbm.at[pl.ds(0, 8)],
                                      grp_v.at[p, j], sems[p]).wait()
                sub = mv[j] % 8
                for h in range(MOVIE_DIM // LANES):
                    comb_v[g * LANES + j, pl.ds(h * LANES, LANES)] = (
                        grp_v[p, j, sub, pl.ds(h * LANES, LANES)])

            @pl.when(k < NG // 2 - 1)
            def _issue_next():
                issue_group(g + 2, p)

        return carry

    lax.fori_loop(0, NG // 2, g2_body, 0)
    pltpu.sync_copy(comb_v, out_hbm.at[pl.ds(base, BPW)])


_movie_call = pl.kernel(
    _movie_body,
    out_type=[jax.ShapeDtypeStruct((B, MOVIE_DIM), jnp.float32)],
    mesh=plsc.VectorSubcoreMesh(core_axis_name="c", subcore_axis_name="s"),
    compiler_params=pltpu.CompilerParams(use_tc_tiling_on_sc=True),
    scratch_types=[
        pltpu.VMEM((NCH, CH), jnp.int32),
        pltpu.VMEM((2, LANES, 8, MOVIE_DIM), jnp.float32),
        pltpu.VMEM((BPW, MOVIE_DIM), jnp.float32),
        pltpu.SemaphoreType.DMA,
        pltpu.SemaphoreType.DMA,
    ],
)


def _cat_bias_body(mid_hbm, mct_hbm, cats_hbm, bias_hbm,
                   out_cat_hbm, out_bias_hbm,
                   mid_v, cats_v, g_v, outc_v, bhi_v, bias_g, bias_c,
                   sem_b, sem_c):
    cid = lax.axis_index("c")
    sid = lax.axis_index("s")
    wid = sid * NC + cid
    base = wid * BPW

    pltpu.sync_copy(mid_hbm.at[wid], mid_v)
    pltpu.sync_copy(mct_hbm.at[wid], cats_v)

    def hi_body(i, carry):
        bhi_v[i // 8, pl.ds((i % 8) * LANES, LANES)] = (
            lax.shift_right_logical(
                mid_v[i // 8, pl.ds((i % 8) * LANES, LANES)], 4))
        return carry

    lax.fori_loop(0, NG, hi_body, 0)

    lane_iota = lax.iota(jnp.int32, LANES)

    for c in range(NCH):
        bias_dma = pltpu.async_copy(
            bias_hbm.at[bhi_v.at[c]], bias_g, sem_b)

        def q_body(q, carry):
            cat_dmas = [
                pltpu.async_copy(
                    cats_hbm.at[cats_v.at[pl.ds((c * L + l) * CH + q * CHC,
                                                CHC)]],
                    g_v.at[l], sem_c)
                for l in range(L)
            ]
            for dma in cat_dmas:
                dma.wait()

            cnt = jnp.zeros((LANES,), jnp.float32)
            for l in range(L):
                cv = cats_v[pl.ds((c * L + l) * CH + q * CHC, LANES)]
                cnt = cnt + jnp.where(cv != 0, 1.0, 0.0)
            recip = 1.0 / jnp.maximum(cnt, 1.0)
            for j in range(LANES):
                rp = recip[j]
                for h in range(CAT_DIM // LANES):
                    s = g_v[0, j, pl.ds(h * LANES, LANES)]
                    for l in range(1, L):
                        s = s + g_v[l, j, pl.ds(h * LANES, LANES)]
                    outc_v[q * CHC + j,
                           pl.ds(h * LANES, LANES)] = s * rp
            return carry

        lax.fori_loop(0, NQ, q_body, 0)
        pltpu.sync_copy(outc_v, out_cat_hbm.at[pl.ds(base + c * CH, CH)])

        bias_dma.wait()

        def bias_body(g, carry):
            lanes16 = mid_v[c, pl.ds(g * LANES, LANES)] & 15
            vals = jnp.zeros((LANES,), jnp.float32)
            for j in range(LANES):
                row = bias_g[g * LANES + j, pl.ds(0, LANES)]
                picked = jnp.take_along_axis(row, lanes16, axis=0)
                vals = jnp.where(lane_iota == j, picked, vals)
            bias_c[pl.ds(g * LANES, LANES)] = vals
            return carry

        lax.fori_loop(0, CH // LANES, bias_body, 0)
        pltpu.sync_copy(bias_c, out_bias_hbm.at[pl.ds(base + c * CH, CH)])


_cat_bias_call = pl.kernel(
    _cat_bias_body,
    out_type=[
        jax.ShapeDtypeStruct((B, CAT_DIM), jnp.float32),
        jax.ShapeDtypeStruct((B,), jnp.float32),
    ],
    mesh=plsc.VectorSubcoreMesh(core_axis_name="c", subcore_axis_name="s"),
    compiler_params=pltpu.CompilerParams(use_tc_tiling_on_sc=False),
    scratch_types=[
        pltpu.VMEM((NCH, CH), jnp.int32),
        pltpu.VMEM((NCH * L * CH,), jnp.int32),
        pltpu.VMEM((L, CHC, CAT_DIM), jnp.float32),
        pltpu.VMEM((CH, CAT_DIM), jnp.float32),
        pltpu.VMEM((NCH, CH), jnp.int32),
        pltpu.VMEM((CH, 16), jnp.float32),
        pltpu.VMEM((CH,), jnp.float32),
        pltpu.SemaphoreType.DMA,
        pltpu.SemaphoreType.DMA,
    ],
)


def kernel(movie_id, movie_categories, emb_movies_W, emb_cats_W, bias_W):
    mid = movie_id.astype(jnp.int32).reshape(NW, NCH, CH)
    mct = (movie_categories.astype(jnp.int32).T
           .reshape(L, NW, NCH, CH).transpose(1, 2, 0, 3)
           .reshape(NW, NCH * L * CH))
    bias16 = bias_W.T.reshape(NUM_MOVIES // 16, 16)
    (mv,) = _movie_call(mid, emb_movies_W)
    cat, bias = _cat_bias_call(mid, mct, emb_cats_W, bias16)
    return jnp.concatenate([mv, cat], axis=1), bias
